# baseline (speedup 1.0000x reference)
import functools

import jax
import jax.numpy as jnp
from jax import lax
from jax.experimental import pallas as pl
from jax.experimental.pallas import tpu as pltpu
from jax.experimental.pallas import tpu_sc as plsc

_CHUNK = 128
_SW = 128


def _edge_msg_body(ea_ref, xs_ref, par_ref, w1_ref, b1_ref, w2_ref, b2m_ref,
                   r_ref, f_ref, out_ref, *, n_valid, base, teb, hd):
    h = jnp.dot(ea_ref[...], w1_ref[...], preferred_element_type=jnp.float32)
    h = jnp.maximum(h + b1_ref[...], 0.0)
    xs = xs_ref[:, :hd]
    q = jnp.dot(h.astype(jnp.bfloat16), w2_ref[...].astype(jnp.bfloat16),
                preferred_element_type=jnp.float32)
    x_exp = jnp.dot(xs.astype(jnp.bfloat16), r_ref[...].astype(jnp.bfloat16),
                    preferred_element_type=jnp.float32)
    p = (q * x_exp).astype(jnp.bfloat16)
    msg = jnp.dot(p, f_ref[...].astype(jnp.bfloat16),
                  preferred_element_type=jnp.float32)
    msg = msg + jnp.dot(xs, b2m_ref[...], preferred_element_type=jnp.float32)
    row = (base + pl.program_id(0) * teb
           + lax.broadcasted_iota(jnp.int32, (teb, 1), 0))
    msg = jnp.where(row < n_valid, msg, 0.0)
    par = par_ref[...]
    out_ref[:, :hd] = msg * (1.0 - par)
    out_ref[:, hd:] = msg * par


def _final_body(pa_ref, pb_ref, x_ref, root_ref, bias_ref, gamma_ref,
                beta_ref, out_ref, *, n, hd):
    agg = (pa_ref[0, :n, :] + pa_ref[1, :n, :]
           + pb_ref[0, :n, :] + pb_ref[1, :n, :])
    pre = agg + jnp.dot(x_ref[...], root_ref[...],
                        preferred_element_type=jnp.float32) + bias_ref[...]
    mean = jnp.mean(pre, axis=0, keepdims=True)
    ctr = pre - mean
    var = jnp.mean(ctr * ctr, axis=0, keepdims=True)
    y = ctr * lax.rsqrt(var + 1e-5) * gamma_ref[...] + beta_ref[...]
    out_ref[...] = x_ref[...] + jnp.maximum(y, 0.0)


def _edge_msgs(ea_p, xs, par, w1, b1r, w2, b2m, rmat, fmat, e, ep, hd, base=0,
               teb=256):
    ed = ea_p.shape[1]
    return pl.pallas_call(
        functools.partial(_edge_msg_body, n_valid=e, base=base, teb=teb,
                          hd=hd),
        grid=(ep // teb,),
        in_specs=[
            pl.BlockSpec((teb, ed), lambda i: (i, 0)),
            pl.BlockSpec((teb, _SW), lambda i: (i, 0)),
            pl.BlockSpec((teb, 1), lambda i: (i, 0)),
            pl.BlockSpec((ed, hd), lambda i: (0, 0)),
            pl.BlockSpec((1, hd), lambda i: (0, 0)),
            pl.BlockSpec((hd, hd * hd), lambda i: (0, 0)),
            pl.BlockSpec((hd, hd), lambda i: (0, 0)),
            pl.BlockSpec((hd, hd * hd), lambda i: (0, 0)),
            pl.BlockSpec((hd * hd, hd), lambda i: (0, 0)),
        ],
        out_specs=pl.BlockSpec((teb, _SW), lambda i: (i, 0)),
        out_shape=jax.ShapeDtypeStruct((ep, _SW), jnp.float32),
    )(ea_p, xs, par, w1, b1r, w2, b2m, rmat, fmat)


def _sc_gather(xpad, idx3, nc, ns):
    nw = nc * ns
    kj = idx3.shape[1]
    epw = kj * _CHUNK
    ep = epw * nw

    @functools.partial(
        pl.kernel,
        mesh=plsc.VectorSubcoreMesh(core_axis_name="c", subcore_axis_name="s"),
        out_type=jax.ShapeDtypeStruct((ep, _SW), jnp.float32),
        scratch_types=[
            pltpu.VMEM((kj, _CHUNK), jnp.int32),
            pltpu.VMEM((epw, _SW), jnp.float32),
            pltpu.SemaphoreType.DMA,
        ],
    )
    def gk(x_hbm, idx_hbm, out_hbm, idx_v, rows_v, sem):
        wid = lax.axis_index("s") * nc + lax.axis_index("c")
        pltpu.sync_copy(idx_hbm.at[wid], idx_v)
        dmas = [
            pltpu.async_copy(x_hbm.at[idx_v.at[j]],
                             rows_v.at[pl.ds(j * _CHUNK, _CHUNK)], sem)
            for j in range(kj)
        ]
        for d in dmas:
            d.wait()
        pltpu.sync_copy(rows_v, out_hbm.at[pl.ds(wid * epw, epw)])

    return gk(xpad, idx3)


def _sc_scatter(msg, idx3, zrows, npad, nc, ns):
    nw = nc * ns
    kj = idx3.shape[1]
    epw = kj * _CHUNK
    rps = npad // ns
    hd = msg.shape[1]

    @functools.partial(
        pl.kernel,
        mesh=plsc.VectorSubcoreMesh(core_axis_name="c", subcore_axis_name="s"),
        out_type=jax.ShapeDtypeStruct((nc, npad, hd), jnp.float32),
        scratch_types=[
            pltpu.VMEM((kj, _CHUNK), jnp.int32),
            pltpu.VMEM((_CHUNK, hd), jnp.float32),
            pltpu.VMEM_SHARED((npad, hd), jnp.float32),
            pltpu.SemaphoreType.DMA,
        ],
    )
    def sk(msg_hbm, idx_hbm, z_hbm, out_hbm, idx_v, msg_v, shared, sem):
        c = lax.axis_index("c")
        s = lax.axis_index("s")
        wid = s * nc + c
        pltpu.sync_copy(z_hbm, shared.at[pl.ds(s * rps, rps)])
        pltpu.sync_copy(idx_hbm.at[wid], idx_v)
        plsc.subcore_barrier()
        for j in range(kj):
            pltpu.sync_copy(msg_hbm.at[pl.ds(wid * epw + j * _CHUNK, _CHUNK)],
                            msg_v)
            pltpu.async_copy(msg_v, shared.at[idx_v.at[j]], sem,
                             add=True).wait()
        plsc.subcore_barrier()
        pltpu.sync_copy(shared.at[pl.ds(s * rps, rps)],
                        out_hbm.at[c].at[pl.ds(s * rps, rps)])

    return sk(msg, idx3, zrows)


def kernel(x, edge_index, edge_attr, W1, b1, W2, b2, root, bias, gamma, beta):
    n, hd = x.shape
    e = edge_attr.shape[0]
    info = plsc.get_sparse_core_info()
    nc, ns = info.num_cores, info.num_subcores
    nw = nc * ns

    epw = (-(-(-(-e // nw)) // _CHUNK)) * _CHUNK
    ep = epw * nw
    pad = ep - e
    src_p = jnp.concatenate([edge_index[0], jnp.zeros((pad,), jnp.int32)])
    dst_p = jnp.concatenate([edge_index[1], jnp.zeros((pad,), jnp.int32)])
    ea_p = jnp.pad(edge_attr, ((0, pad), (0, 0)))

    b2m = b2.reshape(hd, hd)
    eye = jnp.eye(hd, dtype=jnp.float32)
    rmat = jnp.repeat(eye, hd, axis=1)
    fmat = jnp.tile(eye, (hd, 1))
    b1r = b1.reshape(1, hd)
    xpad = jnp.pad(x, ((0, 0), (0, _SW - hd)))

    rps = (-(-(-(-n // 2)) // ns) + 7) // 8 * 8
    np2 = rps * ns
    zrows = jnp.zeros((rps, _SW), jnp.float32)
    dst_pair = dst_p // 2
    par_p = (dst_p % 2).astype(jnp.float32).reshape(ep, 1)

    kj = epw // _CHUNK
    kj_a = -(-kj * 3 // 5)
    ea_cnt = kj_a * _CHUNK * nw
    aggs = []
    for lo, hi in ((0, ea_cnt), (ea_cnt, ep)):
        cnt = hi - lo
        src_i = src_p[lo:hi].reshape(nw, cnt // (nw * _CHUNK), _CHUNK)
        dst_i = dst_pair[lo:hi].reshape(nw, cnt // (nw * _CHUNK), _CHUNK)
        xs_i = _sc_gather(xpad, src_i, nc, ns)
        msg_i = _edge_msgs(ea_p[lo:hi], xs_i, par_p[lo:hi], W1, b1r, W2, b2m,
                           rmat, fmat, e, cnt, hd, base=lo)
        part = _sc_scatter(msg_i, dst_i, zrows, np2, nc, ns)
        aggs.append(part.reshape(nc, 2 * np2, hd))

    return pl.pallas_call(
        functools.partial(_final_body, n=n, hd=hd),
        out_shape=jax.ShapeDtypeStruct((n, hd), jnp.float32),
    )(aggs[0], aggs[1], x, root, bias.reshape(1, hd), gamma.reshape(1, hd),
      beta.reshape(1, hd))


# speedup vs baseline: 1.0297x; 1.0297x over previous
import functools

import jax
import jax.numpy as jnp
from jax import lax
from jax.experimental import pallas as pl
from jax.experimental.pallas import tpu as pltpu
from jax.experimental.pallas import tpu_sc as plsc

_CHUNK = 128
_SW = 128


def _edge_msg_body(ea_ref, xs_ref, par_ref, w1_ref, b1_ref, w2_ref, b2m_ref,
                   r_ref, f_ref, out_ref, *, n_valid, base, teb, hd):
    h = jnp.dot(ea_ref[...], w1_ref[...], preferred_element_type=jnp.float32)
    h = jnp.maximum(h + b1_ref[...], 0.0)
    xs = xs_ref[:, :hd]
    q = jnp.dot(h.astype(jnp.bfloat16), w2_ref[...].astype(jnp.bfloat16),
                preferred_element_type=jnp.float32)
    x_exp = jnp.dot(xs.astype(jnp.bfloat16), r_ref[...].astype(jnp.bfloat16),
                    preferred_element_type=jnp.float32)
    p = (q * x_exp).astype(jnp.bfloat16)
    msg = jnp.dot(p, f_ref[...].astype(jnp.bfloat16),
                  preferred_element_type=jnp.float32)
    msg = msg + jnp.dot(xs, b2m_ref[...], preferred_element_type=jnp.float32)
    row = (base + pl.program_id(0) * teb
           + lax.broadcasted_iota(jnp.int32, (teb, 1), 0))
    msg = jnp.where(row < n_valid, msg, 0.0)
    par = par_ref[...]
    out_ref[:, :hd] = msg * (1.0 - par)
    out_ref[:, hd:] = msg * par


def _final_body(pa_ref, x_ref, root_ref, bias_ref, gamma_ref,
                beta_ref, out_ref, *, n, hd):
    agg = pa_ref[0, :n, :] + pa_ref[1, :n, :]
    pre = agg + jnp.dot(x_ref[...], root_ref[...],
                        preferred_element_type=jnp.float32) + bias_ref[...]
    mean = jnp.mean(pre, axis=0, keepdims=True)
    ctr = pre - mean
    var = jnp.mean(ctr * ctr, axis=0, keepdims=True)
    y = ctr * lax.rsqrt(var + 1e-5) * gamma_ref[...] + beta_ref[...]
    out_ref[...] = x_ref[...] + jnp.maximum(y, 0.0)


def _edge_msgs(ea_p, xs, par, w1, b1r, w2, b2m, rmat, fmat, e, ep, hd, base=0,
               teb=256):
    ed = ea_p.shape[1]
    return pl.pallas_call(
        functools.partial(_edge_msg_body, n_valid=e, base=base, teb=teb,
                          hd=hd),
        grid=(ep // teb,),
        in_specs=[
            pl.BlockSpec((teb, ed), lambda i: (i, 0)),
            pl.BlockSpec((teb, _SW), lambda i: (i, 0)),
            pl.BlockSpec((teb, 1), lambda i: (i, 0)),
            pl.BlockSpec((ed, hd), lambda i: (0, 0)),
            pl.BlockSpec((1, hd), lambda i: (0, 0)),
            pl.BlockSpec((hd, hd * hd), lambda i: (0, 0)),
            pl.BlockSpec((hd, hd), lambda i: (0, 0)),
            pl.BlockSpec((hd, hd * hd), lambda i: (0, 0)),
            pl.BlockSpec((hd * hd, hd), lambda i: (0, 0)),
        ],
        out_specs=pl.BlockSpec((teb, _SW), lambda i: (i, 0)),
        out_shape=jax.ShapeDtypeStruct((ep, _SW), jnp.float32),
    )(ea_p, xs, par, w1, b1r, w2, b2m, rmat, fmat)


def _sc_gather(xpad, idx3, nc, ns):
    nw = nc * ns
    kj = idx3.shape[1]
    epw = kj * _CHUNK
    ep = epw * nw

    @functools.partial(
        pl.kernel,
        mesh=plsc.VectorSubcoreMesh(core_axis_name="c", subcore_axis_name="s"),
        out_type=jax.ShapeDtypeStruct((ep, _SW), jnp.float32),
        scratch_types=[
            pltpu.VMEM((kj, _CHUNK), jnp.int32),
            pltpu.VMEM((epw, _SW), jnp.float32),
            pltpu.SemaphoreType.DMA,
        ],
    )
    def gk(x_hbm, idx_hbm, out_hbm, idx_v, rows_v, sem):
        wid = lax.axis_index("s") * nc + lax.axis_index("c")
        pltpu.sync_copy(idx_hbm.at[wid], idx_v)
        dmas = [
            pltpu.async_copy(x_hbm.at[idx_v.at[j]],
                             rows_v.at[pl.ds(j * _CHUNK, _CHUNK)], sem)
            for j in range(kj)
        ]
        for d in dmas:
            d.wait()
        pltpu.sync_copy(rows_v, out_hbm.at[pl.ds(wid * epw, epw)])

    return gk(xpad, idx3)


def _sc_scatter(msg, idx3, zrows, npad, nc, ns):
    nw = nc * ns
    kj = idx3.shape[1]
    epw = kj * _CHUNK
    rps = npad // ns
    hd = msg.shape[1]

    @functools.partial(
        pl.kernel,
        mesh=plsc.VectorSubcoreMesh(core_axis_name="c", subcore_axis_name="s"),
        out_type=jax.ShapeDtypeStruct((nc, npad, hd), jnp.float32),
        scratch_types=[
            pltpu.VMEM((kj, _CHUNK), jnp.int32),
            pltpu.VMEM((2, _CHUNK, hd), jnp.float32),
            pltpu.VMEM_SHARED((npad, hd), jnp.float32),
            pltpu.SemaphoreType.DMA,
            pltpu.SemaphoreType.DMA,
            pltpu.SemaphoreType.DMA,
        ],
    )
    def sk(msg_hbm, idx_hbm, z_hbm, out_hbm, idx_v, msg_v, shared, sa, s0, s1):
        c = lax.axis_index("c")
        s = lax.axis_index("s")
        wid = s * nc + c
        lsem = (s0, s1)
        pltpu.sync_copy(z_hbm, shared.at[pl.ds(s * rps, rps)])
        pltpu.sync_copy(idx_hbm.at[wid], idx_v)
        plsc.subcore_barrier()
        loads = [pltpu.async_copy(msg_hbm.at[pl.ds(wid * epw, _CHUNK)],
                                  msg_v.at[0], lsem[0])]
        for j in range(kj):
            if j + 1 < kj:
                loads.append(pltpu.async_copy(
                    msg_hbm.at[pl.ds(wid * epw + (j + 1) * _CHUNK, _CHUNK)],
                    msg_v.at[(j + 1) % 2], lsem[(j + 1) % 2]))
            loads[j].wait()
            pltpu.async_copy(msg_v.at[j % 2], shared.at[idx_v.at[j]], sa,
                             add=True).wait()
        plsc.subcore_barrier()
        pltpu.sync_copy(shared.at[pl.ds(s * rps, rps)],
                        out_hbm.at[c].at[pl.ds(s * rps, rps)])

    return sk(msg, idx3, zrows)


def kernel(x, edge_index, edge_attr, W1, b1, W2, b2, root, bias, gamma, beta):
    n, hd = x.shape
    e = edge_attr.shape[0]
    info = plsc.get_sparse_core_info()
    nc, ns = info.num_cores, info.num_subcores
    nw = nc * ns

    epw = (-(-(-(-e // nw)) // _CHUNK)) * _CHUNK
    ep = epw * nw
    pad = ep - e
    src_p = jnp.concatenate([edge_index[0], jnp.zeros((pad,), jnp.int32)])
    dst_p = jnp.concatenate([edge_index[1], jnp.zeros((pad,), jnp.int32)])
    ea_p = jnp.pad(edge_attr, ((0, pad), (0, 0)))

    b2m = b2.reshape(hd, hd)
    eye = jnp.eye(hd, dtype=jnp.float32)
    rmat = jnp.repeat(eye, hd, axis=1)
    fmat = jnp.tile(eye, (hd, 1))
    b1r = b1.reshape(1, hd)
    xpad = jnp.pad(x, ((0, 0), (0, _SW - hd)))

    rps = (-(-(-(-n // 2)) // ns) + 7) // 8 * 8
    np2 = rps * ns
    zrows = jnp.zeros((rps, _SW), jnp.float32)
    dst_pair = dst_p // 2
    par_p = (dst_p % 2).astype(jnp.float32).reshape(ep, 1)

    kj = epw // _CHUNK
    src_i = src_p.reshape(nw, kj, _CHUNK)
    dst_i = dst_pair.reshape(nw, kj, _CHUNK)
    xs = _sc_gather(xpad, src_i, nc, ns)
    msg = _edge_msgs(ea_p, xs, par_p, W1, b1r, W2, b2m, rmat, fmat, e, ep, hd)
    part = _sc_scatter(msg, dst_i, zrows, np2, nc, ns)
    agg = part.reshape(nc, 2 * np2, hd)

    return pl.pallas_call(
        functools.partial(_final_body, n=n, hd=hd),
        out_shape=jax.ShapeDtypeStruct((n, hd), jnp.float32),
    )(agg, x, root, bias.reshape(1, hd), gamma.reshape(1, hd),
      beta.reshape(1, hd))


# speedup vs baseline: 1.0913x; 1.0599x over previous
import functools

import jax
import jax.numpy as jnp
from jax import lax
from jax.experimental import pallas as pl
from jax.experimental.pallas import tpu as pltpu
from jax.experimental.pallas import tpu_sc as plsc

_CHUNK = 128
_SW = 128


def _edge_msg_body(ea_ref, xs_ref, w1_ref, b1_ref, w2_ref, b2m_ref, r_ref,
                   f_ref, out_ref, *, n_valid, base, teb, hd):
    h = jnp.dot(ea_ref[...], w1_ref[...], preferred_element_type=jnp.float32)
    h = jnp.maximum(h + b1_ref[...], 0.0)
    xs = xs_ref[:, :hd]
    q = jnp.dot(h.astype(jnp.bfloat16), w2_ref[...].astype(jnp.bfloat16),
                preferred_element_type=jnp.float32)
    x_exp = jnp.dot(xs.astype(jnp.bfloat16), r_ref[...].astype(jnp.bfloat16),
                    preferred_element_type=jnp.float32)
    p = (q * x_exp).astype(jnp.bfloat16)
    msg = jnp.dot(p, f_ref[...].astype(jnp.bfloat16),
                  preferred_element_type=jnp.float32)
    msg = msg + jnp.dot(xs, b2m_ref[...], preferred_element_type=jnp.float32)
    row = (base + pl.program_id(0) * teb
           + lax.broadcasted_iota(jnp.int32, (teb, 1), 0))
    out_ref[:, :hd] = jnp.where(row < n_valid, msg, 0.0)
    out_ref[:, hd:] = jnp.zeros((teb, _SW - hd), jnp.float32)


def _final_body(pa_ref, pb_ref, x_ref, root_ref, bias_ref, gamma_ref,
                beta_ref, out_ref, *, n, hd):
    agg = pa_ref[:n, :hd] + pb_ref[:n, :hd]
    pre = agg + jnp.dot(x_ref[...], root_ref[...],
                        preferred_element_type=jnp.float32) + bias_ref[...]
    mean = jnp.mean(pre, axis=0, keepdims=True)
    ctr = pre - mean
    var = jnp.mean(ctr * ctr, axis=0, keepdims=True)
    y = ctr * lax.rsqrt(var + 1e-5) * gamma_ref[...] + beta_ref[...]
    out_ref[...] = x_ref[...] + jnp.maximum(y, 0.0)


def _edge_msgs(ea_p, xs, w1, b1r, w2, b2m, rmat, fmat, e, ep, hd, base=0,
               teb=256):
    ed = ea_p.shape[1]
    return pl.pallas_call(
        functools.partial(_edge_msg_body, n_valid=e, base=base, teb=teb,
                          hd=hd),
        grid=(ep // teb,),
        in_specs=[
            pl.BlockSpec((teb, ed), lambda i: (i, 0)),
            pl.BlockSpec((teb, _SW), lambda i: (i, 0)),
            pl.BlockSpec((ed, hd), lambda i: (0, 0)),
            pl.BlockSpec((1, hd), lambda i: (0, 0)),
            pl.BlockSpec((hd, hd * hd), lambda i: (0, 0)),
            pl.BlockSpec((hd, hd), lambda i: (0, 0)),
            pl.BlockSpec((hd, hd * hd), lambda i: (0, 0)),
            pl.BlockSpec((hd * hd, hd), lambda i: (0, 0)),
        ],
        out_specs=pl.BlockSpec((teb, _SW), lambda i: (i, 0)),
        out_shape=jax.ShapeDtypeStruct((ep, _SW), jnp.float32),
    )(ea_p, xs, w1, b1r, w2, b2m, rmat, fmat)


def _sc_gather(xpad, idx3, nc, ns):
    nw = nc * ns
    kj = idx3.shape[1]
    epw = kj * _CHUNK
    ep = epw * nw

    @functools.partial(
        pl.kernel,
        mesh=plsc.VectorSubcoreMesh(core_axis_name="c", subcore_axis_name="s"),
        out_type=jax.ShapeDtypeStruct((ep, _SW), jnp.float32),
        scratch_types=[
            pltpu.VMEM((kj, _CHUNK), jnp.int32),
            pltpu.VMEM((epw, _SW), jnp.float32),
            pltpu.SemaphoreType.DMA,
        ],
    )
    def gk(x_hbm, idx_hbm, out_hbm, idx_v, rows_v, sem):
        wid = lax.axis_index("s") * nc + lax.axis_index("c")
        pltpu.sync_copy(idx_hbm.at[wid], idx_v)
        dmas = [
            pltpu.async_copy(x_hbm.at[idx_v.at[j]],
                             rows_v.at[pl.ds(j * _CHUNK, _CHUNK)], sem)
            for j in range(kj)
        ]
        for d in dmas:
            d.wait()
        pltpu.sync_copy(rows_v, out_hbm.at[pl.ds(wid * epw, epw)])

    return gk(xpad, idx3)


def _sc_scatter(msg, idx2, zrows, npad, nc, ns):
    half = npad // nc
    rps = half // ns
    kj = idx2.shape[1]
    esub = kj * _CHUNK
    groups = _CHUNK // 16

    @functools.partial(
        pl.kernel,
        mesh=plsc.VectorSubcoreMesh(core_axis_name="c", subcore_axis_name="s"),
        out_type=jax.ShapeDtypeStruct((npad, _SW), jnp.float32),
        scratch_types=[
            pltpu.VMEM((kj, _CHUNK), jnp.int32),
            pltpu.VMEM((2, _CHUNK, _SW), jnp.float32),
            pltpu.VMEM_SHARED((half + 8, _SW), jnp.float32),
            pltpu.SemaphoreType.DMA,
            pltpu.SemaphoreType.DMA,
            pltpu.SemaphoreType.DMA,
        ],
    )
    def sk(msg_hbm, idx_hbm, z_hbm, out_hbm, idx_v, msg_v, shared, sa, s0, s1):
        c = lax.axis_index("c")
        s = lax.axis_index("s")
        lsem = (s0, s1)
        pltpu.sync_copy(z_hbm, shared.at[pl.ds(s * rps, rps)])
        pltpu.sync_copy(idx_hbm.at[s], idx_v)
        base = jax.lax.broadcast(c * half, (16,)).astype(jnp.int32)
        trash = jnp.full((16,), half, jnp.int32)
        for j in range(kj):
            for g in range(groups):
                v = idx_v[j, pl.ds(g * 16, 16)]
                local = v - base
                valid = (v >= base) & (local < half)
                idx_v[j, pl.ds(g * 16, 16)] = jnp.where(valid, local, trash)
        plsc.subcore_barrier()
        loads = [pltpu.async_copy(msg_hbm.at[pl.ds(s * esub, _CHUNK)],
                                  msg_v.at[0], lsem[0])]
        for j in range(kj):
            if j + 1 < kj:
                loads.append(pltpu.async_copy(
                    msg_hbm.at[pl.ds(s * esub + (j + 1) * _CHUNK, _CHUNK)],
                    msg_v.at[(j + 1) % 2], lsem[(j + 1) % 2]))
            loads[j].wait()
            pltpu.async_copy(msg_v.at[j % 2], shared.at[idx_v.at[j]], sa,
                             add=True).wait()
        plsc.subcore_barrier()
        pltpu.sync_copy(shared.at[pl.ds(s * rps, rps)],
                        out_hbm.at[pl.ds(c * half + s * rps, rps)])

    return sk(msg, idx2, zrows)


def kernel(x, edge_index, edge_attr, W1, b1, W2, b2, root, bias, gamma, beta):
    n, hd = x.shape
    e = edge_attr.shape[0]
    info = plsc.get_sparse_core_info()
    nc, ns = info.num_cores, info.num_subcores
    nw = nc * ns

    epw = (-(-(-(-e // nw)) // _CHUNK)) * _CHUNK
    ep = epw * nw
    pad = ep - e
    src_p = jnp.concatenate([edge_index[0], jnp.zeros((pad,), jnp.int32)])
    dst_p = jnp.concatenate([edge_index[1], jnp.zeros((pad,), jnp.int32)])
    ea_p = jnp.pad(edge_attr, ((0, pad), (0, 0)))

    b2m = b2.reshape(hd, hd)
    eye = jnp.eye(hd, dtype=jnp.float32)
    rmat = jnp.repeat(eye, hd, axis=1)
    fmat = jnp.tile(eye, (hd, 1))
    b1r = b1.reshape(1, hd)
    xpad = jnp.pad(x, ((0, 0), (0, _SW - hd)))

    rps = (-(-n // (nc * ns)) + 7) // 8 * 8
    npad = rps * nc * ns
    zrows = jnp.zeros((rps, _SW), jnp.float32)

    kj = epw // _CHUNK
    kj_a = -(-kj * 3 // 5)
    ea_cnt = kj_a * _CHUNK * nw
    aggs = []
    for lo, hi in ((0, ea_cnt), (ea_cnt, ep)):
        cnt = hi - lo
        src_i = src_p[lo:hi].reshape(nw, cnt // (nw * _CHUNK), _CHUNK)
        dst_i = dst_p[lo:hi].reshape(ns, cnt // (ns * _CHUNK), _CHUNK)
        xs_i = _sc_gather(xpad, src_i, nc, ns)
        msg_i = _edge_msgs(ea_p[lo:hi], xs_i, W1, b1r, W2, b2m, rmat, fmat,
                           e, cnt, hd, base=lo)
        aggs.append(_sc_scatter(msg_i, dst_i, zrows, npad, nc, ns))

    return pl.pallas_call(
        functools.partial(_final_body, n=n, hd=hd),
        out_shape=jax.ShapeDtypeStruct((n, hd), jnp.float32),
    )(aggs[0], aggs[1], x, root, bias.reshape(1, hd), gamma.reshape(1, hd),
      beta.reshape(1, hd))


# speedup vs baseline: 1.1056x; 1.0131x over previous
import functools

import jax
import jax.numpy as jnp
from jax import lax
from jax.experimental import pallas as pl
from jax.experimental.pallas import tpu as pltpu
from jax.experimental.pallas import tpu_sc as plsc

_CHUNK = 128
_SW = 128


def _edge_msg_body(ea_ref, xs_ref, w1_ref, b1_ref, w2_ref, b2m_ref, r_ref,
                   f_ref, out_ref, *, n_valid, base, teb, hd):
    h = jnp.dot(ea_ref[...], w1_ref[...], preferred_element_type=jnp.float32)
    h = jnp.maximum(h + b1_ref[...], 0.0)
    xs = xs_ref[:, :hd]
    q = jnp.dot(h.astype(jnp.bfloat16), w2_ref[...].astype(jnp.bfloat16),
                preferred_element_type=jnp.float32)
    x_exp = jnp.dot(xs.astype(jnp.bfloat16), r_ref[...].astype(jnp.bfloat16),
                    preferred_element_type=jnp.float32)
    p = (q * x_exp).astype(jnp.bfloat16)
    msg = jnp.dot(p, f_ref[...].astype(jnp.bfloat16),
                  preferred_element_type=jnp.float32)
    msg = msg + jnp.dot(xs, b2m_ref[...], preferred_element_type=jnp.float32)
    row = (base + pl.program_id(0) * teb
           + lax.broadcasted_iota(jnp.int32, (teb, 1), 0))
    out_ref[:, :hd] = jnp.where(row < n_valid, msg, 0.0)
    out_ref[:, hd:] = jnp.zeros((teb, _SW - hd), jnp.float32)


def _final_body(pa_ref, pb_ref, pc_ref, x_ref, root_ref, bias_ref, gamma_ref,
                beta_ref, out_ref, *, n, hd):
    agg = pa_ref[:n, :hd] + pb_ref[:n, :hd] + pc_ref[:n, :hd]
    pre = agg + jnp.dot(x_ref[...], root_ref[...],
                        preferred_element_type=jnp.float32) + bias_ref[...]
    mean = jnp.mean(pre, axis=0, keepdims=True)
    ctr = pre - mean
    var = jnp.mean(ctr * ctr, axis=0, keepdims=True)
    y = ctr * lax.rsqrt(var + 1e-5) * gamma_ref[...] + beta_ref[...]
    out_ref[...] = x_ref[...] + jnp.maximum(y, 0.0)


def _edge_msgs(ea_p, xs, w1, b1r, w2, b2m, rmat, fmat, e, ep, hd, base=0,
               teb=256):
    ed = ea_p.shape[1]
    return pl.pallas_call(
        functools.partial(_edge_msg_body, n_valid=e, base=base, teb=teb,
                          hd=hd),
        grid=(ep // teb,),
        in_specs=[
            pl.BlockSpec((teb, ed), lambda i: (i, 0)),
            pl.BlockSpec((teb, _SW), lambda i: (i, 0)),
            pl.BlockSpec((ed, hd), lambda i: (0, 0)),
            pl.BlockSpec((1, hd), lambda i: (0, 0)),
            pl.BlockSpec((hd, hd * hd), lambda i: (0, 0)),
            pl.BlockSpec((hd, hd), lambda i: (0, 0)),
            pl.BlockSpec((hd, hd * hd), lambda i: (0, 0)),
            pl.BlockSpec((hd * hd, hd), lambda i: (0, 0)),
        ],
        out_specs=pl.BlockSpec((teb, _SW), lambda i: (i, 0)),
        out_shape=jax.ShapeDtypeStruct((ep, _SW), jnp.float32),
    )(ea_p, xs, w1, b1r, w2, b2m, rmat, fmat)


def _sc_gather(xpad, idx3, nc, ns):
    nw = nc * ns
    kj = idx3.shape[1]
    epw = kj * _CHUNK
    ep = epw * nw

    @functools.partial(
        pl.kernel,
        mesh=plsc.VectorSubcoreMesh(core_axis_name="c", subcore_axis_name="s"),
        out_type=jax.ShapeDtypeStruct((ep, _SW), jnp.float32),
        scratch_types=[
            pltpu.VMEM((kj, _CHUNK), jnp.int32),
            pltpu.VMEM((epw, _SW), jnp.float32),
            pltpu.SemaphoreType.DMA,
        ],
    )
    def gk(x_hbm, idx_hbm, out_hbm, idx_v, rows_v, sem):
        wid = lax.axis_index("s") * nc + lax.axis_index("c")
        pltpu.sync_copy(idx_hbm.at[wid], idx_v)
        dmas = [
            pltpu.async_copy(x_hbm.at[idx_v.at[j]],
                             rows_v.at[pl.ds(j * _CHUNK, _CHUNK)], sem)
            for j in range(kj)
        ]
        for d in dmas:
            d.wait()
        pltpu.sync_copy(rows_v, out_hbm.at[pl.ds(wid * epw, epw)])

    return gk(xpad, idx3)


def _sc_scatter(msg, idx2, zrows, npad, nc, ns):
    half = npad // nc
    rps = half // ns
    kj = idx2.shape[1]
    esub = kj * _CHUNK
    groups = _CHUNK // 16

    @functools.partial(
        pl.kernel,
        mesh=plsc.VectorSubcoreMesh(core_axis_name="c", subcore_axis_name="s"),
        out_type=jax.ShapeDtypeStruct((npad, _SW), jnp.float32),
        scratch_types=[
            pltpu.VMEM((kj, _CHUNK), jnp.int32),
            pltpu.VMEM((2, _CHUNK, _SW), jnp.float32),
            pltpu.VMEM_SHARED((half + 8, _SW), jnp.float32),
            pltpu.SemaphoreType.DMA,
            pltpu.SemaphoreType.DMA,
            pltpu.SemaphoreType.DMA,
        ],
    )
    def sk(msg_hbm, idx_hbm, z_hbm, out_hbm, idx_v, msg_v, shared, sa, s0, s1):
        c = lax.axis_index("c")
        s = lax.axis_index("s")
        lsem = (s0, s1)
        pltpu.sync_copy(z_hbm, shared.at[pl.ds(s * rps, rps)])
        pltpu.sync_copy(idx_hbm.at[s], idx_v)
        base = jax.lax.broadcast(c * half, (16,)).astype(jnp.int32)
        trash = jnp.full((16,), half, jnp.int32)
        for j in range(kj):
            for g in range(groups):
                v = idx_v[j, pl.ds(g * 16, 16)]
                local = v - base
                valid = (v >= base) & (local < half)
                idx_v[j, pl.ds(g * 16, 16)] = jnp.where(valid, local, trash)
        plsc.subcore_barrier()
        loads = [pltpu.async_copy(msg_hbm.at[pl.ds(s * esub, _CHUNK)],
                                  msg_v.at[0], lsem[0])]
        for j in range(kj):
            if j + 1 < kj:
                loads.append(pltpu.async_copy(
                    msg_hbm.at[pl.ds(s * esub + (j + 1) * _CHUNK, _CHUNK)],
                    msg_v.at[(j + 1) % 2], lsem[(j + 1) % 2]))
            loads[j].wait()
            pltpu.async_copy(msg_v.at[j % 2], shared.at[idx_v.at[j]], sa,
                             add=True).wait()
        plsc.subcore_barrier()
        pltpu.sync_copy(shared.at[pl.ds(s * rps, rps)],
                        out_hbm.at[pl.ds(c * half + s * rps, rps)])

    return sk(msg, idx2, zrows)


def kernel(x, edge_index, edge_attr, W1, b1, W2, b2, root, bias, gamma, beta):
    n, hd = x.shape
    e = edge_attr.shape[0]
    info = plsc.get_sparse_core_info()
    nc, ns = info.num_cores, info.num_subcores
    nw = nc * ns

    epw = (-(-(-(-e // nw)) // _CHUNK)) * _CHUNK
    ep = epw * nw
    pad = ep - e
    src_p = jnp.concatenate([edge_index[0], jnp.zeros((pad,), jnp.int32)])
    dst_p = jnp.concatenate([edge_index[1], jnp.zeros((pad,), jnp.int32)])
    ea_p = jnp.pad(edge_attr, ((0, pad), (0, 0)))

    b2m = b2.reshape(hd, hd)
    eye = jnp.eye(hd, dtype=jnp.float32)
    rmat = jnp.repeat(eye, hd, axis=1)
    fmat = jnp.tile(eye, (hd, 1))
    b1r = b1.reshape(1, hd)
    xpad = jnp.pad(x, ((0, 0), (0, _SW - hd)))

    rps = (-(-n // (nc * ns)) + 7) // 8 * 8
    npad = rps * nc * ns
    zrows = jnp.zeros((rps, _SW), jnp.float32)

    kj = epw // _CHUNK
    cut1 = 2 * _CHUNK * nw
    cut2 = 4 * _CHUNK * nw
    aggs = []
    for lo, hi in ((0, cut1), (cut1, cut2), (cut2, ep)):
        cnt = hi - lo
        src_i = src_p[lo:hi].reshape(nw, cnt // (nw * _CHUNK), _CHUNK)
        dst_i = dst_p[lo:hi].reshape(ns, cnt // (ns * _CHUNK), _CHUNK)
        xs_i = _sc_gather(xpad, src_i, nc, ns)
        msg_i = _edge_msgs(ea_p[lo:hi], xs_i, W1, b1r, W2, b2m, rmat, fmat,
                           e, cnt, hd, base=lo)
        aggs.append(_sc_scatter(msg_i, dst_i, zrows, npad, nc, ns))

    return pl.pallas_call(
        functools.partial(_final_body, n=n, hd=hd),
        out_shape=jax.ShapeDtypeStruct((n, hd), jnp.float32),
    )(aggs[0], aggs[1], aggs[2], x, root, bias.reshape(1, hd),
      gamma.reshape(1, hd), beta.reshape(1, hd))
